# baseline (speedup 1.0000x reference)
import jax
import jax.numpy as jnp
from jax import lax
from jax.experimental import pallas as pl
from jax.experimental.pallas import tpu as pltpu
from jax.experimental.pallas import tpu_sc as plsc

_MAX_ITERATIONS = 15

_B = 8
_H = 512
_W = 512
_N = _H * _W
_MAXP = 2 * _N
_NS = 16
_NC = 2
_NBPC = _B // _NC
_PB = _N // _NS
_CH = 128
_NCH = _PB // _CH
_GRP = 8
_NGRP = _NCH // _GRP
_R1 = 8
_RF = 4


def _min9(p):
    mn = p[0:_H, 0:_W]
    for dh in range(3):
        for dw in range(3):
            if dh == 0 and dw == 0:
                continue
            mn = jnp.minimum(mn, p[dh:dh + _H, dw:dw + _W])
    return mn


def _prologue_body(x_ref, o_ref):
    xv = x_ref[0]
    row = lax.broadcasted_iota(jnp.int32, (_H, _W), 0)
    col = lax.broadcasted_iota(jnp.int32, (_H, _W), 1)
    val = row * _W + col
    nz = xv > 0
    vertex = (_N * (xv > 1).astype(jnp.int32)
              + _MAXP * (xv == 0).astype(jnp.int32)
              + nz.astype(jnp.int32) * val)
    p = jnp.pad(vertex, ((1, 1), (1, 1)), constant_values=_MAXP)
    o_ref[0] = jnp.where(nz, _min9(p), _MAXP).astype(jnp.int32)


def _stencil_body(xf_ref, xo_ref, o_ref):
    v = xf_ref[0]
    p = jnp.pad(v, ((1, 1), (1, 1)), constant_values=_MAXP)
    o_ref[0] = jnp.where(xo_ref[0] > 0, _min9(p), _MAXP).astype(jnp.int32)


def _tc_prologue(xi):
    return pl.pallas_call(
        _prologue_body,
        grid=(_B,),
        in_specs=[pl.BlockSpec((1, _H, _W), lambda b: (b, 0, 0))],
        out_specs=pl.BlockSpec((1, _H, _W), lambda b: (b, 0, 0)),
        out_shape=jax.ShapeDtypeStruct((_B, _H, _W), jnp.int32),
    )(xi)


def _tc_stencil(xf, xi):
    return pl.pallas_call(
        _stencil_body,
        grid=(_B,),
        in_specs=[pl.BlockSpec((1, _H, _W), lambda b: (b, 0, 0)),
                  pl.BlockSpec((1, _H, _W), lambda b: (b, 0, 0))],
        out_specs=pl.BlockSpec((1, _H, _W), lambda b: (b, 0, 0)),
        out_shape=jax.ShapeDtypeStruct((_B, _H, _W), jnp.int32),
    )(xf, xi)


def _iota16():
    return lax.iota(jnp.int32, 16)


def _gather_chunks(table, idx, dst, sem, idx_is_2d):
    def grp(g, carry):
        cps = []
        for k in range(_GRP):
            j = g * _GRP + k
            off = j * _CH
            if idx_is_2d:
                src = table.at[idx.at[j]]
            else:
                src = table.at[idx.at[pl.ds(off, _CH)]]
            cps.append(pltpu.async_copy(src, dst.at[pl.ds(off, _CH)], sem))
        for cp in cps:
            cp.wait()
        return carry
    lax.fori_loop(0, _NGRP, grp, 0, unroll=False)


def _scatter_chunks(src, ci, table, sem):
    def grp(g, carry):
        cps = []
        for k in range(_GRP):
            j = g * _GRP + k
            off = j * _CH
            cps.append(pltpu.async_copy(
                src.at[pl.ds(off, _CH)], table.at[ci.at[j]], sem))
        for cp in cps:
            cp.wait()
        return carry
    lax.fori_loop(0, _NGRP, grp, 0, unroll=False)


def _dump16(s, off):
    return _N + ((off + s * 1024) & (_PB - 1)) + _iota16()


def _do_round(s, X1, CNT, Bb, D, ci, cbuf, call, sem):
    for _sub in range(2):
        _scatter_chunks(Bb, ci, X1, sem)
        plsc.subcore_barrier()
        _gather_chunks(X1, ci, D, sem, True)

        def crow(i, cnt):
            for k in range(8):
                off = i * _CH + k * 16
                civ = ci[i, pl.ds(k * 16, 16)]
                srcv = Bb[pl.ds(off, 16)]
                gv = D[pl.ds(off, 16)]
                unsat = (civ < _N) & (srcv < gv)
                dump = _dump16(s, off)
                ci[i, pl.ds(k * 16, 16)] = jnp.where(unsat, civ, dump)
                cnt = cnt + jnp.where(unsat, 1, 0).astype(jnp.int32)
            return cnt
        cntv = lax.fori_loop(0, _NCH, crow, jnp.zeros((16,), jnp.int32),
                             unroll=False)
    cbuf[...] = cntv
    pltpu.sync_copy(cbuf, CNT.at[s])
    plsc.subcore_barrier()
    pltpu.sync_copy(CNT, call)
    acc = jnp.zeros((16,), jnp.int32)
    for r in range(_NS):
        acc = acc + call[r, :]
    tot = acc[0]
    for l in range(1, 16):
        tot = tot + acc[l]
    cbuf[...] = jnp.zeros((16,), jnp.int32) + tot


def _rounds(nrounds, s, X1, CNT, Bb, D, ci, cbuf, call, sem):
    cbuf[...] = jnp.zeros((16,), jnp.int32) + 1

    def rbody(r, carry):
        tv = cbuf[...]
        tot = tv[0]

        @pl.when(tot > 0)
        def _():
            _do_round(s, X1, CNT, Bb, D, ci, cbuf, call, sem)
        return carry
    lax.fori_loop(0, nrounds, rbody, 0, unroll=False)


def _s1_body(xf_hbm, x2_hbm, xo_hbm, vv_hbm,
             x1o_hbm, flg_hbm,
             XF, X1, CNT,
             xf_own, xo_own, Bb, D, ci, vv, cbuf, call, sem):
    c = lax.axis_index("c")
    s = lax.axis_index("s")

    def batch_body(bi, carry):
        b = c * _NBPC + bi
        base = s * _PB

        pltpu.sync_copy(xf_hbm.at[b, pl.ds(base, _PB)], xf_own)
        pltpu.sync_copy(x2_hbm.at[b, pl.ds(base, _PB)], D)
        pltpu.sync_copy(xo_hbm.at[b, pl.ds(base, _PB)], xo_own)
        pltpu.sync_copy(vv_hbm.at[b], vv)
        pltpu.sync_copy(xf_own, XF.at[pl.ds(base, _PB)])
        plsc.subcore_barrier()

        validv = vv[...] != 0

        def row1(i, carry):
            for k in range(8):
                off = i * _CH + k * 16
                x2v = D[pl.ds(off, 16)]
                m0v = xo_own[pl.ds(off, 16)] > 0
                adj = jnp.where(x2v >= _N, x2v - _N, x2v)
                adj = jnp.minimum(adj, _N - 1)
                owni = _iota16() + off
                ci[i, pl.ds(k * 16, 16)] = jnp.where(m0v, adj, owni)
            return carry
        lax.fori_loop(0, _NCH, row1, 0, unroll=False)

        _gather_chunks(XF, ci, Bb, sem, True)

        def row2(i, carry):
            for k in range(8):
                off = i * _CH + k * 16
                g1 = Bb[pl.ds(off, 16)]
                m0v = xo_own[pl.ds(off, 16)] > 0
                xfv = xf_own[pl.ds(off, 16)]
                Bb[pl.ds(off, 16)] = jnp.where(m0v, g1, xfv)
                adjx = jnp.where(xfv >= _N, xfv - _N, xfv)
                adjx = jnp.minimum(adjx, _N - 1)
                act = m0v & validv
                dump = _dump16(s, off)
                ci[i, pl.ds(k * 16, 16)] = jnp.where(act, adjx, dump)
            return carry
        lax.fori_loop(0, _NCH, row2, 0, unroll=False)

        pltpu.sync_copy(Bb, X1.at[pl.ds(base, _PB)])
        plsc.subcore_barrier()

        _gather_chunks(X1, ci, D, sem, True)

        def row2b(i, carry):
            for k in range(8):
                off = i * _CH + k * 16
                civ = ci[i, pl.ds(k * 16, 16)]
                srcv = Bb[pl.ds(off, 16)]
                gv = D[pl.ds(off, 16)]
                keep = (civ < _N) & (srcv < gv)
                dump = _dump16(s, off)
                ci[i, pl.ds(k * 16, 16)] = jnp.where(keep, civ, dump)
            return carry
        lax.fori_loop(0, _NCH, row2b, 0, unroll=False)

        _rounds(_R1, s, X1, CNT, Bb, D, ci, cbuf, call, sem)

        pltpu.sync_copy(X1.at[pl.ds(base, _PB)], x1o_hbm.at[b, pl.ds(base, _PB)])
        pltpu.sync_copy(cbuf, flg_hbm.at[b, pl.ds(s * 16, 16)])
        return carry

    lax.fori_loop(0, _NBPC, batch_body, 0, unroll=False)


def _sfix_body(xf_hbm, x2_hbm, xo_hbm, vv_hbm, x1_hbm,
               x1o_hbm, flg_hbm,
               XF, X1, CNT,
               xf_own, xo_own, Bb, D, ci, vv, cbuf, call, sem):
    c = lax.axis_index("c")
    s = lax.axis_index("s")

    def batch_body(bi, carry):
        b = c * _NBPC + bi
        base = s * _PB

        pltpu.sync_copy(xf_hbm.at[b, pl.ds(base, _PB)], xf_own)
        pltpu.sync_copy(x2_hbm.at[b, pl.ds(base, _PB)], D)
        pltpu.sync_copy(xo_hbm.at[b, pl.ds(base, _PB)], xo_own)
        pltpu.sync_copy(vv_hbm.at[b], vv)
        pltpu.sync_copy(xf_own, XF.at[pl.ds(base, _PB)])
        pltpu.sync_copy(x1_hbm.at[b, pl.ds(base, _PB)], X1.at[pl.ds(base, _PB)])
        plsc.subcore_barrier()

        validv = vv[...] != 0

        def row1(i, carry):
            for k in range(8):
                off = i * _CH + k * 16
                x2v = D[pl.ds(off, 16)]
                m0v = xo_own[pl.ds(off, 16)] > 0
                adj = jnp.where(x2v >= _N, x2v - _N, x2v)
                adj = jnp.minimum(adj, _N - 1)
                owni = _iota16() + off
                ci[i, pl.ds(k * 16, 16)] = jnp.where(m0v, adj, owni)
            return carry
        lax.fori_loop(0, _NCH, row1, 0, unroll=False)

        _gather_chunks(XF, ci, Bb, sem, True)

        def row2(i, carry):
            for k in range(8):
                off = i * _CH + k * 16
                g1 = Bb[pl.ds(off, 16)]
                m0v = xo_own[pl.ds(off, 16)] > 0
                xfv = xf_own[pl.ds(off, 16)]
                Bb[pl.ds(off, 16)] = jnp.where(m0v, g1, xfv)
                adjx = jnp.where(xfv >= _N, xfv - _N, xfv)
                adjx = jnp.minimum(adjx, _N - 1)
                act = m0v & validv
                dump = _dump16(s, off)
                ci[i, pl.ds(k * 16, 16)] = jnp.where(act, adjx, dump)
            return carry
        lax.fori_loop(0, _NCH, row2, 0, unroll=False)

        _gather_chunks(X1, ci, D, sem, True)

        def row3(i, carry):
            for k in range(8):
                off = i * _CH + k * 16
                civ = ci[i, pl.ds(k * 16, 16)]
                srcv = Bb[pl.ds(off, 16)]
                gv = D[pl.ds(off, 16)]
                keep = (civ < _N) & (srcv < gv)
                dump = _dump16(s, off)
                ci[i, pl.ds(k * 16, 16)] = jnp.where(keep, civ, dump)
            return carry
        lax.fori_loop(0, _NCH, row3, 0, unroll=False)

        _rounds(_RF, s, X1, CNT, Bb, D, ci, cbuf, call, sem)

        pltpu.sync_copy(X1.at[pl.ds(base, _PB)], x1o_hbm.at[b, pl.ds(base, _PB)])
        pltpu.sync_copy(cbuf, flg_hbm.at[b, pl.ds(s * 16, 16)])
        return carry

    lax.fori_loop(0, _NBPC, batch_body, 0, unroll=False)


def _s2_body(xf_hbm, xo_hbm, x1_hbm,
             xfo_hbm, tsk_hbm,
             X1, xf_own, xo_own, ci, Bb, D, cbuf, sem):
    c = lax.axis_index("c")
    s = lax.axis_index("s")

    def batch_body(bi, carry):
        b = c * _NBPC + bi
        base = s * _PB

        pltpu.sync_copy(xf_hbm.at[b, pl.ds(base, _PB)], xf_own)
        pltpu.sync_copy(xo_hbm.at[b, pl.ds(base, _PB)], xo_own)
        pltpu.sync_copy(x1_hbm.at[b, pl.ds(base, _PB)], D)
        pltpu.sync_copy(D, X1.at[pl.ds(base, _PB)])
        plsc.subcore_barrier()

        def row1(i, carry):
            for k in range(8):
                off = i * _CH + k * 16
                xh = D[pl.ds(off, 16)]
                m0v = xo_own[pl.ds(off, 16)] > 0
                adjh = jnp.where(xh >= _N, xh - _N, xh)
                adjh = jnp.minimum(adjh, _N - 1)
                owni = _iota16() + off
                ci[i, pl.ds(k * 16, 16)] = jnp.where(m0v, adjh, owni)
            return carry
        lax.fori_loop(0, _NCH, row1, 0, unroll=False)

        _gather_chunks(X1, ci, Bb, sem, True)

        def row2(i, ts):
            for k in range(8):
                off = i * _CH + k * 16
                m0v = xo_own[pl.ds(off, 16)] > 0
                out = jnp.where(m0v, Bb[pl.ds(off, 16)], D[pl.ds(off, 16)])
                D[pl.ds(off, 16)] = out
                diff = jnp.abs(xf_own[pl.ds(off, 16)] - out)
                ts = ts + jnp.where(m0v, diff, 0).astype(jnp.int32)
            return ts
        tsv = lax.fori_loop(0, _NCH, row2, jnp.zeros((16,), jnp.int32),
                            unroll=False)

        pltpu.sync_copy(D, xfo_hbm.at[b, pl.ds(base, _PB)])
        cbuf[...] = tsv
        pltpu.sync_copy(cbuf, tsk_hbm.at[b, pl.ds(s * 16, 16)])
        return carry

    lax.fori_loop(0, _NBPC, batch_body, 0, unroll=False)


def _mesh():
    return plsc.VectorSubcoreMesh(core_axis_name="c", subcore_axis_name="s")


_SC_SCRATCH = [
    pltpu.VMEM_SHARED((_N,), jnp.int32),
    pltpu.VMEM_SHARED((_N + _PB + 16,), jnp.int32),
    pltpu.VMEM_SHARED((_NS, 16), jnp.int32),
    pltpu.VMEM((_PB,), jnp.int32),
    pltpu.VMEM((_PB,), jnp.int32),
    pltpu.VMEM((_PB,), jnp.int32),
    pltpu.VMEM((_PB,), jnp.int32),
    pltpu.VMEM((_NCH, _CH), jnp.int32),
    pltpu.VMEM((16,), jnp.int32),
    pltpu.VMEM((16,), jnp.int32),
    pltpu.VMEM((_NS, 16), jnp.int32),
    pltpu.SemaphoreType.DMA,
]


def _s1(xf, x2f, xo, vv):
    f = pl.kernel(
        _s1_body,
        out_type=(jax.ShapeDtypeStruct((_B, _N), jnp.int32),
                  jax.ShapeDtypeStruct((_B, _NS * 16), jnp.int32)),
        mesh=_mesh(),
        scratch_types=list(_SC_SCRATCH),
    )
    return f(xf, x2f, xo, vv)


def _sfix(xf, x2f, xo, vv, x1h):
    f = pl.kernel(
        _sfix_body,
        out_type=(jax.ShapeDtypeStruct((_B, _N), jnp.int32),
                  jax.ShapeDtypeStruct((_B, _NS * 16), jnp.int32)),
        mesh=_mesh(),
        scratch_types=list(_SC_SCRATCH),
    )
    return f(xf, x2f, xo, vv, x1h)


def _s2(xf, xo, x1h):
    f = pl.kernel(
        _s2_body,
        out_type=(jax.ShapeDtypeStruct((_B, _N), jnp.int32),
                  jax.ShapeDtypeStruct((_B, _NS * 16), jnp.int32)),
        mesh=_mesh(),
        scratch_types=[
            pltpu.VMEM_SHARED((_N,), jnp.int32),
            pltpu.VMEM((_PB,), jnp.int32),
            pltpu.VMEM((_PB,), jnp.int32),
            pltpu.VMEM((_NCH, _CH), jnp.int32),
            pltpu.VMEM((_PB,), jnp.int32),
            pltpu.VMEM((_PB,), jnp.int32),
            pltpu.VMEM((16,), jnp.int32),
            pltpu.SemaphoreType.DMA,
        ],
    )
    return f(xf, xo, x1h)


def kernel(x):
    x = x.astype(jnp.int32)
    B, C, H, W = x.shape
    xi = x.reshape(B, H, W)
    xf = _tc_prologue(xi).reshape(B, _N)
    xo = xi.reshape(B, _N)
    T_skip = jnp.ones((B,), dtype=jnp.int32)
    cum = jnp.ones((B,), dtype=jnp.int32)
    it = jnp.array(_MAX_ITERATIONS, dtype=jnp.int32)

    def loop_cond(carry):
        xf, cum, T_skip, it = carry
        return (jnp.sum(T_skip) > 0) & (it > 0)

    def loop_body(carry):
        xf, cum, T_skip, it = carry
        it = it - 1
        cum = cum & (T_skip > 0).astype(jnp.int32)
        x2f = _tc_stencil(xf.reshape(B, H, W), xi).reshape(B, _N)
        vv = jnp.broadcast_to(cum[:, None], (B, 16)).astype(jnp.int32)
        x1h, flg = _s1(xf, x2f, xo, vv)

        def fix_cond(c):
            return jnp.sum(jnp.max(c[1], axis=1)) > 0

        def fix_body(c):
            return _sfix(xf, x2f, xo, vv, c[0])

        x1h, flg = lax.while_loop(fix_cond, fix_body, (x1h, flg))
        xf2, tsk = _s2(xf, xo, x1h)
        T_skip = jnp.sum(tsk, axis=1).astype(jnp.int32)
        return xf2, cum, T_skip, it

    xf, cum, T_skip, it = jax.lax.while_loop(
        loop_cond, loop_body, (xf, cum, T_skip, it))
    return xf.reshape(B, 1, H, W)


# speedup vs baseline: 1.0397x; 1.0397x over previous
import jax
import jax.numpy as jnp
from jax import lax
from jax.experimental import pallas as pl
from jax.experimental.pallas import tpu as pltpu
from jax.experimental.pallas import tpu_sc as plsc

_MAX_ITERATIONS = 15

_B = 8
_H = 512
_W = 512
_N = _H * _W
_MAXP = 2 * _N
_NS = 16
_NC = 2
_NBPC = _B // _NC
_PB = _N // _NS
_CH = 128
_NCH = _PB // _CH
_GRP = 8
_NGRP = _NCH // _GRP
_R1 = 8
_RF = 4


def _min9(p):
    mn = p[0:_H, 0:_W]
    for dh in range(3):
        for dw in range(3):
            if dh == 0 and dw == 0:
                continue
            mn = jnp.minimum(mn, p[dh:dh + _H, dw:dw + _W])
    return mn


def _prologue_body(x_ref, o_ref):
    xv = x_ref[0]
    row = lax.broadcasted_iota(jnp.int32, (_H, _W), 0)
    col = lax.broadcasted_iota(jnp.int32, (_H, _W), 1)
    val = row * _W + col
    nz = xv > 0
    vertex = (_N * (xv > 1).astype(jnp.int32)
              + _MAXP * (xv == 0).astype(jnp.int32)
              + nz.astype(jnp.int32) * val)
    p = jnp.pad(vertex, ((1, 1), (1, 1)), constant_values=_MAXP)
    o_ref[0] = jnp.where(nz, _min9(p), _MAXP).astype(jnp.int32)


def _stencil_body(xf_ref, xo_ref, o_ref):
    v = xf_ref[0]
    p = jnp.pad(v, ((1, 1), (1, 1)), constant_values=_MAXP)
    o_ref[0] = jnp.where(xo_ref[0] > 0, _min9(p), _MAXP).astype(jnp.int32)


def _tc_prologue(xi):
    return pl.pallas_call(
        _prologue_body,
        grid=(_B,),
        in_specs=[pl.BlockSpec((1, _H, _W), lambda b: (b, 0, 0))],
        out_specs=pl.BlockSpec((1, _H, _W), lambda b: (b, 0, 0)),
        out_shape=jax.ShapeDtypeStruct((_B, _H, _W), jnp.int32),
    )(xi)


def _tc_stencil(xf, xi):
    return pl.pallas_call(
        _stencil_body,
        grid=(_B,),
        in_specs=[pl.BlockSpec((1, _H, _W), lambda b: (b, 0, 0)),
                  pl.BlockSpec((1, _H, _W), lambda b: (b, 0, 0))],
        out_specs=pl.BlockSpec((1, _H, _W), lambda b: (b, 0, 0)),
        out_shape=jax.ShapeDtypeStruct((_B, _H, _W), jnp.int32),
    )(xf, xi)


def _iota16():
    return lax.iota(jnp.int32, 16)


def _gather_chunks(table, idx, dst, sem, idx_is_2d):
    def grp(g, carry):
        cps = []
        for k in range(_GRP):
            j = g * _GRP + k
            off = j * _CH
            if idx_is_2d:
                src = table.at[idx.at[j]]
            else:
                src = table.at[idx.at[pl.ds(off, _CH)]]
            cps.append(pltpu.async_copy(src, dst.at[pl.ds(off, _CH)], sem))
        for cp in cps:
            cp.wait()
        return carry
    lax.fori_loop(0, _NGRP, grp, 0, unroll=False)


def _scatter_chunks(src, ci, table, sem):
    def grp(g, carry):
        cps = []
        for k in range(_GRP):
            j = g * _GRP + k
            off = j * _CH
            cps.append(pltpu.async_copy(
                src.at[pl.ds(off, _CH)], table.at[ci.at[j]], sem))
        for cp in cps:
            cp.wait()
        return carry
    lax.fori_loop(0, _NGRP, grp, 0, unroll=False)


def _dump16(s, off):
    return _N + ((off + s * 1024) & (_PB - 1)) + _iota16()


def _do_round(s, X1, CNT, Bb, D, ci, cbuf, call, pbuf, sem):
    locv = pbuf[...]
    loc = locv[0]

    @pl.when(loc > 0)
    def _():
        _scatter_chunks(Bb, ci, X1, sem)
    plsc.subcore_barrier()

    @pl.when(loc > 0)
    def _():
        _gather_chunks(X1, ci, D, sem, True)

        def crow(i, cnt):
            for k in range(8):
                off = i * _CH + k * 16
                civ = ci[i, pl.ds(k * 16, 16)]
                srcv = Bb[pl.ds(off, 16)]
                gv = D[pl.ds(off, 16)]
                unsat = (civ < _N) & (srcv < gv)
                dump = _dump16(s, off)
                ci[i, pl.ds(k * 16, 16)] = jnp.where(unsat, civ, dump)
                cnt = cnt + jnp.where(unsat, 1, 0).astype(jnp.int32)
            return cnt
        cntv = lax.fori_loop(0, _NCH, crow, jnp.zeros((16,), jnp.int32),
                             unroll=False)
        locn = cntv[0]
        for l in range(1, 16):
            locn = locn + cntv[l]
        pbuf[...] = jnp.zeros((16,), jnp.int32) + locn
        cbuf[...] = cntv
        pltpu.sync_copy(cbuf, CNT.at[s])

    @pl.when(loc == 0)
    def _():
        cbuf[...] = jnp.zeros((16,), jnp.int32)
        pltpu.sync_copy(cbuf, CNT.at[s])

    plsc.subcore_barrier()
    pltpu.sync_copy(CNT, call)
    acc = jnp.zeros((16,), jnp.int32)
    for r in range(_NS):
        acc = acc + call[r, :]
    tot = acc[0]
    for l in range(1, 16):
        tot = tot + acc[l]
    cbuf[...] = jnp.zeros((16,), jnp.int32) + tot


def _rounds(nrounds, s, X1, CNT, Bb, D, ci, cbuf, call, pbuf, sem):
    cbuf[...] = jnp.zeros((16,), jnp.int32) + 1

    def rbody(r, carry):
        tv = cbuf[...]
        tot = tv[0]

        @pl.when(tot > 0)
        def _():
            _do_round(s, X1, CNT, Bb, D, ci, cbuf, call, pbuf, sem)
        return carry
    lax.fori_loop(0, nrounds, rbody, 0, unroll=False)


def _s1_body(xf_hbm, x2_hbm, xo_hbm, vv_hbm,
             x1o_hbm, flg_hbm,
             XF, X1, CNT,
             xf_own, xo_own, Bb, D, ci, vv, cbuf, call, pbuf, sem):
    c = lax.axis_index("c")
    s = lax.axis_index("s")

    def batch_body(bi, carry):
        b = c * _NBPC + bi
        base = s * _PB

        pltpu.sync_copy(xf_hbm.at[b, pl.ds(base, _PB)], xf_own)
        pltpu.sync_copy(x2_hbm.at[b, pl.ds(base, _PB)], D)
        pltpu.sync_copy(xo_hbm.at[b, pl.ds(base, _PB)], xo_own)
        pltpu.sync_copy(vv_hbm.at[b], vv)
        pltpu.sync_copy(xf_own, XF.at[pl.ds(base, _PB)])
        plsc.subcore_barrier()

        validv = vv[...] != 0

        def row1(i, carry):
            for k in range(8):
                off = i * _CH + k * 16
                x2v = D[pl.ds(off, 16)]
                m0v = xo_own[pl.ds(off, 16)] > 0
                adj = jnp.where(x2v >= _N, x2v - _N, x2v)
                adj = jnp.minimum(adj, _N - 1)
                owni = _iota16() + off
                ci[i, pl.ds(k * 16, 16)] = jnp.where(m0v, adj, owni)
            return carry
        lax.fori_loop(0, _NCH, row1, 0, unroll=False)

        _gather_chunks(XF, ci, Bb, sem, True)

        def row2(i, carry):
            for k in range(8):
                off = i * _CH + k * 16
                g1 = Bb[pl.ds(off, 16)]
                m0v = xo_own[pl.ds(off, 16)] > 0
                xfv = xf_own[pl.ds(off, 16)]
                Bb[pl.ds(off, 16)] = jnp.where(m0v, g1, xfv)
                adjx = jnp.where(xfv >= _N, xfv - _N, xfv)
                adjx = jnp.minimum(adjx, _N - 1)
                act = m0v & validv
                dump = _dump16(s, off)
                ci[i, pl.ds(k * 16, 16)] = jnp.where(act, adjx, dump)
            return carry
        lax.fori_loop(0, _NCH, row2, 0, unroll=False)

        pltpu.sync_copy(Bb, X1.at[pl.ds(base, _PB)])
        plsc.subcore_barrier()

        _gather_chunks(X1, ci, D, sem, True)

        def row2b(i, carry):
            for k in range(8):
                off = i * _CH + k * 16
                civ = ci[i, pl.ds(k * 16, 16)]
                srcv = Bb[pl.ds(off, 16)]
                gv = D[pl.ds(off, 16)]
                keep = (civ < _N) & (srcv < gv)
                dump = _dump16(s, off)
                ci[i, pl.ds(k * 16, 16)] = jnp.where(keep, civ, dump)
                carry = carry + jnp.where(keep, 1, 0).astype(jnp.int32)
            return carry
        pk = lax.fori_loop(0, _NCH, row2b, jnp.zeros((16,), jnp.int32),
                           unroll=False)
        loc0 = pk[0]
        for l in range(1, 16):
            loc0 = loc0 + pk[l]
        pbuf[...] = jnp.zeros((16,), jnp.int32) + loc0

        _rounds(_R1, s, X1, CNT, Bb, D, ci, cbuf, call, pbuf, sem)

        pltpu.sync_copy(X1.at[pl.ds(base, _PB)], x1o_hbm.at[b, pl.ds(base, _PB)])
        pltpu.sync_copy(cbuf, flg_hbm.at[b, pl.ds(s * 16, 16)])
        return carry

    lax.fori_loop(0, _NBPC, batch_body, 0, unroll=False)


def _sfix_body(xf_hbm, x2_hbm, xo_hbm, vv_hbm, x1_hbm,
               x1o_hbm, flg_hbm,
               XF, X1, CNT,
               xf_own, xo_own, Bb, D, ci, vv, cbuf, call, pbuf, sem):
    c = lax.axis_index("c")
    s = lax.axis_index("s")

    def batch_body(bi, carry):
        b = c * _NBPC + bi
        base = s * _PB

        pltpu.sync_copy(xf_hbm.at[b, pl.ds(base, _PB)], xf_own)
        pltpu.sync_copy(x2_hbm.at[b, pl.ds(base, _PB)], D)
        pltpu.sync_copy(xo_hbm.at[b, pl.ds(base, _PB)], xo_own)
        pltpu.sync_copy(vv_hbm.at[b], vv)
        pltpu.sync_copy(xf_own, XF.at[pl.ds(base, _PB)])
        pltpu.sync_copy(x1_hbm.at[b, pl.ds(base, _PB)], X1.at[pl.ds(base, _PB)])
        plsc.subcore_barrier()

        validv = vv[...] != 0

        def row1(i, carry):
            for k in range(8):
                off = i * _CH + k * 16
                x2v = D[pl.ds(off, 16)]
                m0v = xo_own[pl.ds(off, 16)] > 0
                adj = jnp.where(x2v >= _N, x2v - _N, x2v)
                adj = jnp.minimum(adj, _N - 1)
                owni = _iota16() + off
                ci[i, pl.ds(k * 16, 16)] = jnp.where(m0v, adj, owni)
            return carry
        lax.fori_loop(0, _NCH, row1, 0, unroll=False)

        _gather_chunks(XF, ci, Bb, sem, True)

        def row2(i, carry):
            for k in range(8):
                off = i * _CH + k * 16
                g1 = Bb[pl.ds(off, 16)]
                m0v = xo_own[pl.ds(off, 16)] > 0
                xfv = xf_own[pl.ds(off, 16)]
                Bb[pl.ds(off, 16)] = jnp.where(m0v, g1, xfv)
                adjx = jnp.where(xfv >= _N, xfv - _N, xfv)
                adjx = jnp.minimum(adjx, _N - 1)
                act = m0v & validv
                dump = _dump16(s, off)
                ci[i, pl.ds(k * 16, 16)] = jnp.where(act, adjx, dump)
            return carry
        lax.fori_loop(0, _NCH, row2, 0, unroll=False)

        _gather_chunks(X1, ci, D, sem, True)

        def row3(i, carry):
            for k in range(8):
                off = i * _CH + k * 16
                civ = ci[i, pl.ds(k * 16, 16)]
                srcv = Bb[pl.ds(off, 16)]
                gv = D[pl.ds(off, 16)]
                keep = (civ < _N) & (srcv < gv)
                dump = _dump16(s, off)
                ci[i, pl.ds(k * 16, 16)] = jnp.where(keep, civ, dump)
                carry = carry + jnp.where(keep, 1, 0).astype(jnp.int32)
            return carry
        pk = lax.fori_loop(0, _NCH, row3, jnp.zeros((16,), jnp.int32),
                           unroll=False)
        loc0 = pk[0]
        for l in range(1, 16):
            loc0 = loc0 + pk[l]
        pbuf[...] = jnp.zeros((16,), jnp.int32) + loc0

        _rounds(_RF, s, X1, CNT, Bb, D, ci, cbuf, call, pbuf, sem)

        pltpu.sync_copy(X1.at[pl.ds(base, _PB)], x1o_hbm.at[b, pl.ds(base, _PB)])
        pltpu.sync_copy(cbuf, flg_hbm.at[b, pl.ds(s * 16, 16)])
        return carry

    lax.fori_loop(0, _NBPC, batch_body, 0, unroll=False)


def _s2_body(xf_hbm, xo_hbm, x1_hbm,
             xfo_hbm, tsk_hbm,
             X1, xf_own, xo_own, ci, Bb, D, cbuf, sem):
    c = lax.axis_index("c")
    s = lax.axis_index("s")

    def batch_body(bi, carry):
        b = c * _NBPC + bi
        base = s * _PB

        pltpu.sync_copy(xf_hbm.at[b, pl.ds(base, _PB)], xf_own)
        pltpu.sync_copy(xo_hbm.at[b, pl.ds(base, _PB)], xo_own)
        pltpu.sync_copy(x1_hbm.at[b, pl.ds(base, _PB)], D)
        pltpu.sync_copy(D, X1.at[pl.ds(base, _PB)])
        plsc.subcore_barrier()

        def row1(i, carry):
            for k in range(8):
                off = i * _CH + k * 16
                xh = D[pl.ds(off, 16)]
                m0v = xo_own[pl.ds(off, 16)] > 0
                adjh = jnp.where(xh >= _N, xh - _N, xh)
                adjh = jnp.minimum(adjh, _N - 1)
                owni = _iota16() + off
                ci[i, pl.ds(k * 16, 16)] = jnp.where(m0v, adjh, owni)
            return carry
        lax.fori_loop(0, _NCH, row1, 0, unroll=False)

        _gather_chunks(X1, ci, Bb, sem, True)

        def row2(i, ts):
            for k in range(8):
                off = i * _CH + k * 16
                m0v = xo_own[pl.ds(off, 16)] > 0
                out = jnp.where(m0v, Bb[pl.ds(off, 16)], D[pl.ds(off, 16)])
                D[pl.ds(off, 16)] = out
                diff = jnp.abs(xf_own[pl.ds(off, 16)] - out)
                ts = ts + jnp.where(m0v, diff, 0).astype(jnp.int32)
            return ts
        tsv = lax.fori_loop(0, _NCH, row2, jnp.zeros((16,), jnp.int32),
                            unroll=False)

        pltpu.sync_copy(D, xfo_hbm.at[b, pl.ds(base, _PB)])
        cbuf[...] = tsv
        pltpu.sync_copy(cbuf, tsk_hbm.at[b, pl.ds(s * 16, 16)])
        return carry

    lax.fori_loop(0, _NBPC, batch_body, 0, unroll=False)


def _mesh():
    return plsc.VectorSubcoreMesh(core_axis_name="c", subcore_axis_name="s")


_SC_SCRATCH = [
    pltpu.VMEM_SHARED((_N,), jnp.int32),
    pltpu.VMEM_SHARED((_N + _PB + 16,), jnp.int32),
    pltpu.VMEM_SHARED((_NS, 16), jnp.int32),
    pltpu.VMEM((_PB,), jnp.int32),
    pltpu.VMEM((_PB,), jnp.int32),
    pltpu.VMEM((_PB,), jnp.int32),
    pltpu.VMEM((_PB,), jnp.int32),
    pltpu.VMEM((_NCH, _CH), jnp.int32),
    pltpu.VMEM((16,), jnp.int32),
    pltpu.VMEM((16,), jnp.int32),
    pltpu.VMEM((_NS, 16), jnp.int32),
    pltpu.VMEM((16,), jnp.int32),
    pltpu.SemaphoreType.DMA,
]


def _s1(xf, x2f, xo, vv):
    f = pl.kernel(
        _s1_body,
        out_type=(jax.ShapeDtypeStruct((_B, _N), jnp.int32),
                  jax.ShapeDtypeStruct((_B, _NS * 16), jnp.int32)),
        mesh=_mesh(),
        scratch_types=list(_SC_SCRATCH),
    )
    return f(xf, x2f, xo, vv)


def _sfix(xf, x2f, xo, vv, x1h):
    f = pl.kernel(
        _sfix_body,
        out_type=(jax.ShapeDtypeStruct((_B, _N), jnp.int32),
                  jax.ShapeDtypeStruct((_B, _NS * 16), jnp.int32)),
        mesh=_mesh(),
        scratch_types=list(_SC_SCRATCH),
    )
    return f(xf, x2f, xo, vv, x1h)


def _s2(xf, xo, x1h):
    f = pl.kernel(
        _s2_body,
        out_type=(jax.ShapeDtypeStruct((_B, _N), jnp.int32),
                  jax.ShapeDtypeStruct((_B, _NS * 16), jnp.int32)),
        mesh=_mesh(),
        scratch_types=[
            pltpu.VMEM_SHARED((_N,), jnp.int32),
            pltpu.VMEM((_PB,), jnp.int32),
            pltpu.VMEM((_PB,), jnp.int32),
            pltpu.VMEM((_NCH, _CH), jnp.int32),
            pltpu.VMEM((_PB,), jnp.int32),
            pltpu.VMEM((_PB,), jnp.int32),
            pltpu.VMEM((16,), jnp.int32),
            pltpu.SemaphoreType.DMA,
        ],
    )
    return f(xf, xo, x1h)


def kernel(x):
    x = x.astype(jnp.int32)
    B, C, H, W = x.shape
    xi = x.reshape(B, H, W)
    xf = _tc_prologue(xi).reshape(B, _N)
    xo = xi.reshape(B, _N)
    T_skip = jnp.ones((B,), dtype=jnp.int32)
    cum = jnp.ones((B,), dtype=jnp.int32)
    it = jnp.array(_MAX_ITERATIONS, dtype=jnp.int32)

    def loop_cond(carry):
        xf, cum, T_skip, it = carry
        return (jnp.sum(T_skip) > 0) & (it > 0)

    def loop_body(carry):
        xf, cum, T_skip, it = carry
        it = it - 1
        cum = cum & (T_skip > 0).astype(jnp.int32)
        x2f = _tc_stencil(xf.reshape(B, H, W), xi).reshape(B, _N)
        vv = jnp.broadcast_to(cum[:, None], (B, 16)).astype(jnp.int32)
        x1h, flg = _s1(xf, x2f, xo, vv)

        def fix_cond(c):
            return jnp.sum(jnp.max(c[1], axis=1)) > 0

        def fix_body(c):
            return _sfix(xf, x2f, xo, vv, c[0])

        x1h, flg = lax.while_loop(fix_cond, fix_body, (x1h, flg))
        xf2, tsk = _s2(xf, xo, x1h)
        T_skip = jnp.sum(tsk, axis=1).astype(jnp.int32)
        return xf2, cum, T_skip, it

    xf, cum, T_skip, it = jax.lax.while_loop(
        loop_cond, loop_body, (xf, cum, T_skip, it))
    return xf.reshape(B, 1, H, W)


# speedup vs baseline: 1.0986x; 1.0567x over previous
import jax
import jax.numpy as jnp
from jax import lax
from jax.experimental import pallas as pl
from jax.experimental.pallas import tpu as pltpu
from jax.experimental.pallas import tpu_sc as plsc

_MAX_ITERATIONS = 15

_B = 8
_H = 512
_W = 512
_N = _H * _W
_MAXP = 2 * _N
_NS = 16
_NC = 2
_NBPC = _B // _NC
_PB = _N // _NS
_CH = 128
_NCH = _PB // _CH
_GRP = 16
_NGRP = _NCH // _GRP
_R1 = 8
_RF = 4


def _min9(p):
    mn = p[0:_H, 0:_W]
    for dh in range(3):
        for dw in range(3):
            if dh == 0 and dw == 0:
                continue
            mn = jnp.minimum(mn, p[dh:dh + _H, dw:dw + _W])
    return mn


def _prologue_body(x_ref, o_ref):
    xv = x_ref[0]
    row = lax.broadcasted_iota(jnp.int32, (_H, _W), 0)
    col = lax.broadcasted_iota(jnp.int32, (_H, _W), 1)
    val = row * _W + col
    nz = xv > 0
    vertex = (_N * (xv > 1).astype(jnp.int32)
              + _MAXP * (xv == 0).astype(jnp.int32)
              + nz.astype(jnp.int32) * val)
    p = jnp.pad(vertex, ((1, 1), (1, 1)), constant_values=_MAXP)
    o_ref[0] = jnp.where(nz, _min9(p), _MAXP).astype(jnp.int32)


def _stencil_body(xf_ref, xo_ref, o_ref):
    v = xf_ref[0]
    p = jnp.pad(v, ((1, 1), (1, 1)), constant_values=_MAXP)
    o_ref[0] = jnp.where(xo_ref[0] > 0, _min9(p), _MAXP).astype(jnp.int32)


def _tc_prologue(xi):
    return pl.pallas_call(
        _prologue_body,
        grid=(_B,),
        in_specs=[pl.BlockSpec((1, _H, _W), lambda b: (b, 0, 0))],
        out_specs=pl.BlockSpec((1, _H, _W), lambda b: (b, 0, 0)),
        out_shape=jax.ShapeDtypeStruct((_B, _H, _W), jnp.int32),
    )(xi)


def _tc_stencil(xf, xi):
    return pl.pallas_call(
        _stencil_body,
        grid=(_B,),
        in_specs=[pl.BlockSpec((1, _H, _W), lambda b: (b, 0, 0)),
                  pl.BlockSpec((1, _H, _W), lambda b: (b, 0, 0))],
        out_specs=pl.BlockSpec((1, _H, _W), lambda b: (b, 0, 0)),
        out_shape=jax.ShapeDtypeStruct((_B, _H, _W), jnp.int32),
    )(xf, xi)


def _iota16():
    return lax.iota(jnp.int32, 16)


def _gather_chunks(table, idx, dst, sem, idx_is_2d):
    def grp(g, carry):
        cps = []
        for k in range(_GRP):
            j = g * _GRP + k
            off = j * _CH
            if idx_is_2d:
                src = table.at[idx.at[j]]
            else:
                src = table.at[idx.at[pl.ds(off, _CH)]]
            cps.append(pltpu.async_copy(src, dst.at[pl.ds(off, _CH)], sem))
        for cp in cps:
            cp.wait()
        return carry
    lax.fori_loop(0, _NGRP, grp, 0, unroll=False)


def _scatter_chunks(src, ci, table, sem):
    def grp(g, carry):
        cps = []
        for k in range(_GRP):
            j = g * _GRP + k
            off = j * _CH
            cps.append(pltpu.async_copy(
                src.at[pl.ds(off, _CH)], table.at[ci.at[j]], sem))
        for cp in cps:
            cp.wait()
        return carry
    lax.fori_loop(0, _NGRP, grp, 0, unroll=False)


def _dump16(s, off):
    return _N + ((off + s * 1024) & (_PB - 1)) + _iota16()


def _do_round(s, X1, CNT, Bb, D, ci, cbuf, call, pbuf, sem):
    locv = pbuf[...]
    loc = locv[0]

    @pl.when(loc > 0)
    def _():
        _scatter_chunks(Bb, ci, X1, sem)
    plsc.subcore_barrier()

    @pl.when(loc > 0)
    def _():
        _gather_chunks(X1, ci, D, sem, True)

        def crow(i, cnt):
            for k in range(8):
                off = i * _CH + k * 16
                civ = ci[i, pl.ds(k * 16, 16)]
                srcv = Bb[pl.ds(off, 16)]
                gv = D[pl.ds(off, 16)]
                unsat = (civ < _N) & (srcv < gv)
                dump = _dump16(s, off)
                ci[i, pl.ds(k * 16, 16)] = jnp.where(unsat, civ, dump)
                cnt = cnt + jnp.where(unsat, 1, 0).astype(jnp.int32)
            return cnt
        cntv = lax.fori_loop(0, _NCH, crow, jnp.zeros((16,), jnp.int32),
                             unroll=False)
        locn = cntv[0]
        for l in range(1, 16):
            locn = locn + cntv[l]
        pbuf[...] = jnp.zeros((16,), jnp.int32) + locn
        cbuf[...] = cntv
        pltpu.sync_copy(cbuf, CNT.at[s])

    @pl.when(loc == 0)
    def _():
        cbuf[...] = jnp.zeros((16,), jnp.int32)
        pltpu.sync_copy(cbuf, CNT.at[s])

    plsc.subcore_barrier()
    pltpu.sync_copy(CNT, call)
    acc = jnp.zeros((16,), jnp.int32)
    for r in range(_NS):
        acc = acc + call[r, :]
    tot = acc[0]
    for l in range(1, 16):
        tot = tot + acc[l]
    cbuf[...] = jnp.zeros((16,), jnp.int32) + tot


def _rounds(nrounds, s, X1, CNT, Bb, D, ci, cbuf, call, pbuf, sem):
    cbuf[...] = jnp.zeros((16,), jnp.int32) + 1

    def rbody(r, carry):
        tv = cbuf[...]
        tot = tv[0]

        @pl.when(tot > 0)
        def _():
            _do_round(s, X1, CNT, Bb, D, ci, cbuf, call, pbuf, sem)
        return carry
    lax.fori_loop(0, nrounds, rbody, 0, unroll=False)


def _s1_body(xf_hbm, x2_hbm, xo_hbm, vv_hbm,
             x1o_hbm, flg_hbm,
             XF, X1, CNT,
             xf_own, xo_own, Bb, D, ci, vv, cbuf, call, pbuf, sem):
    c = lax.axis_index("c")
    s = lax.axis_index("s")

    def batch_body(bi, carry):
        b = c * _NBPC + bi
        base = s * _PB

        pltpu.sync_copy(xf_hbm.at[b, pl.ds(base, _PB)], xf_own)
        pltpu.sync_copy(x2_hbm.at[b, pl.ds(base, _PB)], D)
        pltpu.sync_copy(xo_hbm.at[b, pl.ds(base, _PB)], xo_own)
        pltpu.sync_copy(vv_hbm.at[b], vv)
        pltpu.sync_copy(xf_own, XF.at[pl.ds(base, _PB)])
        plsc.subcore_barrier()

        validv = vv[...] != 0

        def row1(i, carry):
            for k in range(8):
                off = i * _CH + k * 16
                x2v = D[pl.ds(off, 16)]
                m0v = xo_own[pl.ds(off, 16)] > 0
                adj = jnp.where(x2v >= _N, x2v - _N, x2v)
                adj = jnp.minimum(adj, _N - 1)
                owni = _iota16() + off
                ci[i, pl.ds(k * 16, 16)] = jnp.where(m0v, adj, owni)
            return carry
        lax.fori_loop(0, _NCH, row1, 0, unroll=False)

        _gather_chunks(XF, ci, Bb, sem, True)

        def row2(i, carry):
            for k in range(8):
                off = i * _CH + k * 16
                g1 = Bb[pl.ds(off, 16)]
                m0v = xo_own[pl.ds(off, 16)] > 0
                xfv = xf_own[pl.ds(off, 16)]
                Bb[pl.ds(off, 16)] = jnp.where(m0v, g1, xfv)
                adjx = jnp.where(xfv >= _N, xfv - _N, xfv)
                adjx = jnp.minimum(adjx, _N - 1)
                act = m0v & validv
                dump = _dump16(s, off)
                ci[i, pl.ds(k * 16, 16)] = jnp.where(act, adjx, dump)
            return carry
        lax.fori_loop(0, _NCH, row2, 0, unroll=False)

        pltpu.sync_copy(Bb, X1.at[pl.ds(base, _PB)])
        plsc.subcore_barrier()

        _gather_chunks(X1, ci, D, sem, True)

        def row2b(i, carry):
            for k in range(8):
                off = i * _CH + k * 16
                civ = ci[i, pl.ds(k * 16, 16)]
                srcv = Bb[pl.ds(off, 16)]
                gv = D[pl.ds(off, 16)]
                keep = (civ < _N) & (srcv < gv)
                dump = _dump16(s, off)
                ci[i, pl.ds(k * 16, 16)] = jnp.where(keep, civ, dump)
                carry = carry + jnp.where(keep, 1, 0).astype(jnp.int32)
            return carry
        pk = lax.fori_loop(0, _NCH, row2b, jnp.zeros((16,), jnp.int32),
                           unroll=False)
        loc0 = pk[0]
        for l in range(1, 16):
            loc0 = loc0 + pk[l]
        pbuf[...] = jnp.zeros((16,), jnp.int32) + loc0

        _rounds(_R1, s, X1, CNT, Bb, D, ci, cbuf, call, pbuf, sem)

        pltpu.sync_copy(X1.at[pl.ds(base, _PB)], x1o_hbm.at[b, pl.ds(base, _PB)])
        pltpu.sync_copy(cbuf, flg_hbm.at[b, pl.ds(s * 16, 16)])
        return carry

    lax.fori_loop(0, _NBPC, batch_body, 0, unroll=False)


def _sfix_body(xf_hbm, x2_hbm, xo_hbm, vv_hbm, x1_hbm,
               x1o_hbm, flg_hbm,
               XF, X1, CNT,
               xf_own, xo_own, Bb, D, ci, vv, cbuf, call, pbuf, sem):
    c = lax.axis_index("c")
    s = lax.axis_index("s")

    def batch_body(bi, carry):
        b = c * _NBPC + bi
        base = s * _PB

        pltpu.sync_copy(xf_hbm.at[b, pl.ds(base, _PB)], xf_own)
        pltpu.sync_copy(x2_hbm.at[b, pl.ds(base, _PB)], D)
        pltpu.sync_copy(xo_hbm.at[b, pl.ds(base, _PB)], xo_own)
        pltpu.sync_copy(vv_hbm.at[b], vv)
        pltpu.sync_copy(xf_own, XF.at[pl.ds(base, _PB)])
        pltpu.sync_copy(x1_hbm.at[b, pl.ds(base, _PB)], X1.at[pl.ds(base, _PB)])
        plsc.subcore_barrier()

        validv = vv[...] != 0

        def row1(i, carry):
            for k in range(8):
                off = i * _CH + k * 16
                x2v = D[pl.ds(off, 16)]
                m0v = xo_own[pl.ds(off, 16)] > 0
                adj = jnp.where(x2v >= _N, x2v - _N, x2v)
                adj = jnp.minimum(adj, _N - 1)
                owni = _iota16() + off
                ci[i, pl.ds(k * 16, 16)] = jnp.where(m0v, adj, owni)
            return carry
        lax.fori_loop(0, _NCH, row1, 0, unroll=False)

        _gather_chunks(XF, ci, Bb, sem, True)

        def row2(i, carry):
            for k in range(8):
                off = i * _CH + k * 16
                g1 = Bb[pl.ds(off, 16)]
                m0v = xo_own[pl.ds(off, 16)] > 0
                xfv = xf_own[pl.ds(off, 16)]
                Bb[pl.ds(off, 16)] = jnp.where(m0v, g1, xfv)
                adjx = jnp.where(xfv >= _N, xfv - _N, xfv)
                adjx = jnp.minimum(adjx, _N - 1)
                act = m0v & validv
                dump = _dump16(s, off)
                ci[i, pl.ds(k * 16, 16)] = jnp.where(act, adjx, dump)
            return carry
        lax.fori_loop(0, _NCH, row2, 0, unroll=False)

        _gather_chunks(X1, ci, D, sem, True)

        def row3(i, carry):
            for k in range(8):
                off = i * _CH + k * 16
                civ = ci[i, pl.ds(k * 16, 16)]
                srcv = Bb[pl.ds(off, 16)]
                gv = D[pl.ds(off, 16)]
                keep = (civ < _N) & (srcv < gv)
                dump = _dump16(s, off)
                ci[i, pl.ds(k * 16, 16)] = jnp.where(keep, civ, dump)
                carry = carry + jnp.where(keep, 1, 0).astype(jnp.int32)
            return carry
        pk = lax.fori_loop(0, _NCH, row3, jnp.zeros((16,), jnp.int32),
                           unroll=False)
        loc0 = pk[0]
        for l in range(1, 16):
            loc0 = loc0 + pk[l]
        pbuf[...] = jnp.zeros((16,), jnp.int32) + loc0

        _rounds(_RF, s, X1, CNT, Bb, D, ci, cbuf, call, pbuf, sem)

        pltpu.sync_copy(X1.at[pl.ds(base, _PB)], x1o_hbm.at[b, pl.ds(base, _PB)])
        pltpu.sync_copy(cbuf, flg_hbm.at[b, pl.ds(s * 16, 16)])
        return carry

    lax.fori_loop(0, _NBPC, batch_body, 0, unroll=False)


def _s2_body(xf_hbm, xo_hbm, x1_hbm,
             xfo_hbm, tsk_hbm,
             X1, xf_own, xo_own, ci, Bb, D, cbuf, sem):
    c = lax.axis_index("c")
    s = lax.axis_index("s")

    def batch_body(bi, carry):
        b = c * _NBPC + bi
        base = s * _PB

        pltpu.sync_copy(xf_hbm.at[b, pl.ds(base, _PB)], xf_own)
        pltpu.sync_copy(xo_hbm.at[b, pl.ds(base, _PB)], xo_own)
        pltpu.sync_copy(x1_hbm.at[b, pl.ds(base, _PB)], D)
        pltpu.sync_copy(D, X1.at[pl.ds(base, _PB)])
        plsc.subcore_barrier()

        def row1(i, carry):
            for k in range(8):
                off = i * _CH + k * 16
                xh = D[pl.ds(off, 16)]
                m0v = xo_own[pl.ds(off, 16)] > 0
                adjh = jnp.where(xh >= _N, xh - _N, xh)
                adjh = jnp.minimum(adjh, _N - 1)
                owni = _iota16() + off
                ci[i, pl.ds(k * 16, 16)] = jnp.where(m0v, adjh, owni)
            return carry
        lax.fori_loop(0, _NCH, row1, 0, unroll=False)

        _gather_chunks(X1, ci, Bb, sem, True)

        def row2(i, ts):
            for k in range(8):
                off = i * _CH + k * 16
                m0v = xo_own[pl.ds(off, 16)] > 0
                out = jnp.where(m0v, Bb[pl.ds(off, 16)], D[pl.ds(off, 16)])
                D[pl.ds(off, 16)] = out
                diff = jnp.abs(xf_own[pl.ds(off, 16)] - out)
                ts = ts + jnp.where(m0v, diff, 0).astype(jnp.int32)
            return ts
        tsv = lax.fori_loop(0, _NCH, row2, jnp.zeros((16,), jnp.int32),
                            unroll=False)

        pltpu.sync_copy(D, xfo_hbm.at[b, pl.ds(base, _PB)])
        cbuf[...] = tsv
        pltpu.sync_copy(cbuf, tsk_hbm.at[b, pl.ds(s * 16, 16)])
        return carry

    lax.fori_loop(0, _NBPC, batch_body, 0, unroll=False)


def _mesh():
    return plsc.VectorSubcoreMesh(core_axis_name="c", subcore_axis_name="s")


_SC_SCRATCH = [
    pltpu.VMEM_SHARED((_N,), jnp.int32),
    pltpu.VMEM_SHARED((_N + _PB + 16,), jnp.int32),
    pltpu.VMEM_SHARED((_NS, 16), jnp.int32),
    pltpu.VMEM((_PB,), jnp.int32),
    pltpu.VMEM((_PB,), jnp.int32),
    pltpu.VMEM((_PB,), jnp.int32),
    pltpu.VMEM((_PB,), jnp.int32),
    pltpu.VMEM((_NCH, _CH), jnp.int32),
    pltpu.VMEM((16,), jnp.int32),
    pltpu.VMEM((16,), jnp.int32),
    pltpu.VMEM((_NS, 16), jnp.int32),
    pltpu.VMEM((16,), jnp.int32),
    pltpu.SemaphoreType.DMA,
]


def _s1(xf, x2f, xo, vv):
    f = pl.kernel(
        _s1_body,
        out_type=(jax.ShapeDtypeStruct((_B, _N), jnp.int32),
                  jax.ShapeDtypeStruct((_B, _NS * 16), jnp.int32)),
        mesh=_mesh(),
        scratch_types=list(_SC_SCRATCH),
    )
    return f(xf, x2f, xo, vv)


def _sfix(xf, x2f, xo, vv, x1h):
    f = pl.kernel(
        _sfix_body,
        out_type=(jax.ShapeDtypeStruct((_B, _N), jnp.int32),
                  jax.ShapeDtypeStruct((_B, _NS * 16), jnp.int32)),
        mesh=_mesh(),
        scratch_types=list(_SC_SCRATCH),
    )
    return f(xf, x2f, xo, vv, x1h)


def _s2(xf, xo, x1h):
    f = pl.kernel(
        _s2_body,
        out_type=(jax.ShapeDtypeStruct((_B, _N), jnp.int32),
                  jax.ShapeDtypeStruct((_B, _NS * 16), jnp.int32)),
        mesh=_mesh(),
        scratch_types=[
            pltpu.VMEM_SHARED((_N,), jnp.int32),
            pltpu.VMEM((_PB,), jnp.int32),
            pltpu.VMEM((_PB,), jnp.int32),
            pltpu.VMEM((_NCH, _CH), jnp.int32),
            pltpu.VMEM((_PB,), jnp.int32),
            pltpu.VMEM((_PB,), jnp.int32),
            pltpu.VMEM((16,), jnp.int32),
            pltpu.SemaphoreType.DMA,
        ],
    )
    return f(xf, xo, x1h)


def kernel(x):
    x = x.astype(jnp.int32)
    B, C, H, W = x.shape
    xi = x.reshape(B, H, W)
    xf = _tc_prologue(xi).reshape(B, _N)
    xo = xi.reshape(B, _N)
    T_skip = jnp.ones((B,), dtype=jnp.int32)
    cum = jnp.ones((B,), dtype=jnp.int32)
    it = jnp.array(_MAX_ITERATIONS, dtype=jnp.int32)

    def loop_cond(carry):
        xf, cum, T_skip, it = carry
        return (jnp.sum(T_skip) > 0) & (it > 0)

    def loop_body(carry):
        xf, cum, T_skip, it = carry
        it = it - 1
        cum = cum & (T_skip > 0).astype(jnp.int32)
        x2f = _tc_stencil(xf.reshape(B, H, W), xi).reshape(B, _N)
        vv = jnp.broadcast_to(cum[:, None], (B, 16)).astype(jnp.int32)
        x1h, flg = _s1(xf, x2f, xo, vv)

        def fix_cond(c):
            return jnp.sum(jnp.max(c[1], axis=1)) > 0

        def fix_body(c):
            return _sfix(xf, x2f, xo, vv, c[0])

        x1h, flg = lax.while_loop(fix_cond, fix_body, (x1h, flg))
        xf2, tsk = _s2(xf, xo, x1h)
        T_skip = jnp.sum(tsk, axis=1).astype(jnp.int32)
        return xf2, cum, T_skip, it

    xf, cum, T_skip, it = jax.lax.while_loop(
        loop_cond, loop_body, (xf, cum, T_skip, it))
    return xf.reshape(B, 1, H, W)


# speedup vs baseline: 1.1213x; 1.0206x over previous
import jax
import jax.numpy as jnp
from jax import lax
from jax.experimental import pallas as pl
from jax.experimental.pallas import tpu as pltpu
from jax.experimental.pallas import tpu_sc as plsc

_MAX_ITERATIONS = 15

_B = 8
_H = 512
_W = 512
_N = _H * _W
_MAXP = 2 * _N
_NS = 16
_NC = 2
_NBPC = _B // _NC
_PB = _N // _NS
_CH = 128
_NCH = _PB // _CH
_GRP = 32
_NGRP = _NCH // _GRP
_R1 = 8
_RF = 4


def _min9(p):
    mn = p[0:_H, 0:_W]
    for dh in range(3):
        for dw in range(3):
            if dh == 0 and dw == 0:
                continue
            mn = jnp.minimum(mn, p[dh:dh + _H, dw:dw + _W])
    return mn


def _prologue_body(x_ref, o_ref):
    xv = x_ref[0]
    row = lax.broadcasted_iota(jnp.int32, (_H, _W), 0)
    col = lax.broadcasted_iota(jnp.int32, (_H, _W), 1)
    val = row * _W + col
    nz = xv > 0
    vertex = (_N * (xv > 1).astype(jnp.int32)
              + _MAXP * (xv == 0).astype(jnp.int32)
              + nz.astype(jnp.int32) * val)
    p = jnp.pad(vertex, ((1, 1), (1, 1)), constant_values=_MAXP)
    o_ref[0] = jnp.where(nz, _min9(p), _MAXP).astype(jnp.int32)


def _stencil_body(xf_ref, xo_ref, o_ref):
    v = xf_ref[0]
    p = jnp.pad(v, ((1, 1), (1, 1)), constant_values=_MAXP)
    o_ref[0] = jnp.where(xo_ref[0] > 0, _min9(p), _MAXP).astype(jnp.int32)


def _tc_prologue(xi):
    return pl.pallas_call(
        _prologue_body,
        grid=(_B,),
        in_specs=[pl.BlockSpec((1, _H, _W), lambda b: (b, 0, 0))],
        out_specs=pl.BlockSpec((1, _H, _W), lambda b: (b, 0, 0)),
        out_shape=jax.ShapeDtypeStruct((_B, _H, _W), jnp.int32),
    )(xi)


def _tc_stencil(xf, xi):
    return pl.pallas_call(
        _stencil_body,
        grid=(_B,),
        in_specs=[pl.BlockSpec((1, _H, _W), lambda b: (b, 0, 0)),
                  pl.BlockSpec((1, _H, _W), lambda b: (b, 0, 0))],
        out_specs=pl.BlockSpec((1, _H, _W), lambda b: (b, 0, 0)),
        out_shape=jax.ShapeDtypeStruct((_B, _H, _W), jnp.int32),
    )(xf, xi)


def _iota16():
    return lax.iota(jnp.int32, 16)


def _gather_chunks(table, idx, dst, sem, idx_is_2d):
    def grp(g, carry):
        cps = []
        for k in range(_GRP):
            j = g * _GRP + k
            off = j * _CH
            if idx_is_2d:
                src = table.at[idx.at[j]]
            else:
                src = table.at[idx.at[pl.ds(off, _CH)]]
            cps.append(pltpu.async_copy(src, dst.at[pl.ds(off, _CH)], sem))
        for cp in cps:
            cp.wait()
        return carry
    lax.fori_loop(0, _NGRP, grp, 0, unroll=False)


def _scatter_chunks(src, ci, table, sem):
    def grp(g, carry):
        cps = []
        for k in range(_GRP):
            j = g * _GRP + k
            off = j * _CH
            cps.append(pltpu.async_copy(
                src.at[pl.ds(off, _CH)], table.at[ci.at[j]], sem))
        for cp in cps:
            cp.wait()
        return carry
    lax.fori_loop(0, _NGRP, grp, 0, unroll=False)


def _dump16(s, off):
    return _N + ((off + s * 1024) & (_PB - 1)) + _iota16()


def _do_round(s, X1, CNT, Bb, D, ci, cbuf, call, pbuf, sem):
    locv = pbuf[...]
    loc = locv[0]

    @pl.when(loc > 0)
    def _():
        _scatter_chunks(Bb, ci, X1, sem)
    plsc.subcore_barrier()

    @pl.when(loc > 0)
    def _():
        _gather_chunks(X1, ci, D, sem, True)

        def crow(i, cnt):
            for k in range(8):
                off = i * _CH + k * 16
                civ = ci[i, pl.ds(k * 16, 16)]
                srcv = Bb[pl.ds(off, 16)]
                gv = D[pl.ds(off, 16)]
                unsat = (civ < _N) & (srcv < gv)
                dump = _dump16(s, off)
                ci[i, pl.ds(k * 16, 16)] = jnp.where(unsat, civ, dump)
                cnt = cnt + jnp.where(unsat, 1, 0).astype(jnp.int32)
            return cnt
        cntv = lax.fori_loop(0, _NCH, crow, jnp.zeros((16,), jnp.int32),
                             unroll=False)
        locn = cntv[0]
        for l in range(1, 16):
            locn = locn + cntv[l]
        pbuf[...] = jnp.zeros((16,), jnp.int32) + locn
        cbuf[...] = cntv
        pltpu.sync_copy(cbuf, CNT.at[s])

    @pl.when(loc == 0)
    def _():
        cbuf[...] = jnp.zeros((16,), jnp.int32)
        pltpu.sync_copy(cbuf, CNT.at[s])

    plsc.subcore_barrier()
    pltpu.sync_copy(CNT, call)
    acc = jnp.zeros((16,), jnp.int32)
    for r in range(_NS):
        acc = acc + call[r, :]
    tot = acc[0]
    for l in range(1, 16):
        tot = tot + acc[l]
    cbuf[...] = jnp.zeros((16,), jnp.int32) + tot


def _rounds(nrounds, s, X1, CNT, Bb, D, ci, cbuf, call, pbuf, sem):
    cbuf[...] = jnp.zeros((16,), jnp.int32) + 1

    def rbody(r, carry):
        tv = cbuf[...]
        tot = tv[0]

        @pl.when(tot > 0)
        def _():
            _do_round(s, X1, CNT, Bb, D, ci, cbuf, call, pbuf, sem)
        return carry
    lax.fori_loop(0, nrounds, rbody, 0, unroll=False)


def _s1_body(xf_hbm, x2_hbm, xo_hbm, vv_hbm,
             x1o_hbm, flg_hbm,
             XF, X1, CNT,
             xf_own, xo_own, Bb, D, ci, vv, cbuf, call, pbuf, sem):
    c = lax.axis_index("c")
    s = lax.axis_index("s")

    def batch_body(bi, carry):
        b = c * _NBPC + bi
        base = s * _PB

        pltpu.sync_copy(xf_hbm.at[b, pl.ds(base, _PB)], xf_own)
        pltpu.sync_copy(x2_hbm.at[b, pl.ds(base, _PB)], D)
        pltpu.sync_copy(xo_hbm.at[b, pl.ds(base, _PB)], xo_own)
        pltpu.sync_copy(vv_hbm.at[b], vv)
        pltpu.sync_copy(xf_own, XF.at[pl.ds(base, _PB)])
        plsc.subcore_barrier()

        validv = vv[...] != 0

        def row1(i, carry):
            for k in range(8):
                off = i * _CH + k * 16
                x2v = D[pl.ds(off, 16)]
                m0v = xo_own[pl.ds(off, 16)] > 0
                adj = jnp.where(x2v >= _N, x2v - _N, x2v)
                adj = jnp.minimum(adj, _N - 1)
                owni = _iota16() + off
                ci[i, pl.ds(k * 16, 16)] = jnp.where(m0v, adj, owni)
            return carry
        lax.fori_loop(0, _NCH, row1, 0, unroll=False)

        _gather_chunks(XF, ci, Bb, sem, True)

        def row2(i, carry):
            for k in range(8):
                off = i * _CH + k * 16
                g1 = Bb[pl.ds(off, 16)]
                m0v = xo_own[pl.ds(off, 16)] > 0
                xfv = xf_own[pl.ds(off, 16)]
                Bb[pl.ds(off, 16)] = jnp.where(m0v, g1, xfv)
                adjx = jnp.where(xfv >= _N, xfv - _N, xfv)
                adjx = jnp.minimum(adjx, _N - 1)
                act = m0v & validv
                dump = _dump16(s, off)
                ci[i, pl.ds(k * 16, 16)] = jnp.where(act, adjx, dump)
            return carry
        lax.fori_loop(0, _NCH, row2, 0, unroll=False)

        pltpu.sync_copy(Bb, X1.at[pl.ds(base, _PB)])
        plsc.subcore_barrier()

        _gather_chunks(X1, ci, D, sem, True)

        def row2b(i, carry):
            for k in range(8):
                off = i * _CH + k * 16
                civ = ci[i, pl.ds(k * 16, 16)]
                srcv = Bb[pl.ds(off, 16)]
                gv = D[pl.ds(off, 16)]
                keep = (civ < _N) & (srcv < gv)
                dump = _dump16(s, off)
                ci[i, pl.ds(k * 16, 16)] = jnp.where(keep, civ, dump)
                carry = carry + jnp.where(keep, 1, 0).astype(jnp.int32)
            return carry
        pk = lax.fori_loop(0, _NCH, row2b, jnp.zeros((16,), jnp.int32),
                           unroll=False)
        loc0 = pk[0]
        for l in range(1, 16):
            loc0 = loc0 + pk[l]
        pbuf[...] = jnp.zeros((16,), jnp.int32) + loc0

        _rounds(_R1, s, X1, CNT, Bb, D, ci, cbuf, call, pbuf, sem)

        pltpu.sync_copy(X1.at[pl.ds(base, _PB)], x1o_hbm.at[b, pl.ds(base, _PB)])
        pltpu.sync_copy(cbuf, flg_hbm.at[b, pl.ds(s * 16, 16)])
        return carry

    lax.fori_loop(0, _NBPC, batch_body, 0, unroll=False)


def _sfix_body(xf_hbm, x2_hbm, xo_hbm, vv_hbm, x1_hbm,
               x1o_hbm, flg_hbm,
               XF, X1, CNT,
               xf_own, xo_own, Bb, D, ci, vv, cbuf, call, pbuf, sem):
    c = lax.axis_index("c")
    s = lax.axis_index("s")

    def batch_body(bi, carry):
        b = c * _NBPC + bi
        base = s * _PB

        pltpu.sync_copy(xf_hbm.at[b, pl.ds(base, _PB)], xf_own)
        pltpu.sync_copy(x2_hbm.at[b, pl.ds(base, _PB)], D)
        pltpu.sync_copy(xo_hbm.at[b, pl.ds(base, _PB)], xo_own)
        pltpu.sync_copy(vv_hbm.at[b], vv)
        pltpu.sync_copy(xf_own, XF.at[pl.ds(base, _PB)])
        pltpu.sync_copy(x1_hbm.at[b, pl.ds(base, _PB)], X1.at[pl.ds(base, _PB)])
        plsc.subcore_barrier()

        validv = vv[...] != 0

        def row1(i, carry):
            for k in range(8):
                off = i * _CH + k * 16
                x2v = D[pl.ds(off, 16)]
                m0v = xo_own[pl.ds(off, 16)] > 0
                adj = jnp.where(x2v >= _N, x2v - _N, x2v)
                adj = jnp.minimum(adj, _N - 1)
                owni = _iota16() + off
                ci[i, pl.ds(k * 16, 16)] = jnp.where(m0v, adj, owni)
            return carry
        lax.fori_loop(0, _NCH, row1, 0, unroll=False)

        _gather_chunks(XF, ci, Bb, sem, True)

        def row2(i, carry):
            for k in range(8):
                off = i * _CH + k * 16
                g1 = Bb[pl.ds(off, 16)]
                m0v = xo_own[pl.ds(off, 16)] > 0
                xfv = xf_own[pl.ds(off, 16)]
                Bb[pl.ds(off, 16)] = jnp.where(m0v, g1, xfv)
                adjx = jnp.where(xfv >= _N, xfv - _N, xfv)
                adjx = jnp.minimum(adjx, _N - 1)
                act = m0v & validv
                dump = _dump16(s, off)
                ci[i, pl.ds(k * 16, 16)] = jnp.where(act, adjx, dump)
            return carry
        lax.fori_loop(0, _NCH, row2, 0, unroll=False)

        _gather_chunks(X1, ci, D, sem, True)

        def row3(i, carry):
            for k in range(8):
                off = i * _CH + k * 16
                civ = ci[i, pl.ds(k * 16, 16)]
                srcv = Bb[pl.ds(off, 16)]
                gv = D[pl.ds(off, 16)]
                keep = (civ < _N) & (srcv < gv)
                dump = _dump16(s, off)
                ci[i, pl.ds(k * 16, 16)] = jnp.where(keep, civ, dump)
                carry = carry + jnp.where(keep, 1, 0).astype(jnp.int32)
            return carry
        pk = lax.fori_loop(0, _NCH, row3, jnp.zeros((16,), jnp.int32),
                           unroll=False)
        loc0 = pk[0]
        for l in range(1, 16):
            loc0 = loc0 + pk[l]
        pbuf[...] = jnp.zeros((16,), jnp.int32) + loc0

        _rounds(_RF, s, X1, CNT, Bb, D, ci, cbuf, call, pbuf, sem)

        pltpu.sync_copy(X1.at[pl.ds(base, _PB)], x1o_hbm.at[b, pl.ds(base, _PB)])
        pltpu.sync_copy(cbuf, flg_hbm.at[b, pl.ds(s * 16, 16)])
        return carry

    lax.fori_loop(0, _NBPC, batch_body, 0, unroll=False)


def _s2_body(xf_hbm, xo_hbm, x1_hbm,
             xfo_hbm, tsk_hbm,
             X1, xf_own, xo_own, ci, Bb, D, cbuf, sem):
    c = lax.axis_index("c")
    s = lax.axis_index("s")

    def batch_body(bi, carry):
        b = c * _NBPC + bi
        base = s * _PB

        pltpu.sync_copy(xf_hbm.at[b, pl.ds(base, _PB)], xf_own)
        pltpu.sync_copy(xo_hbm.at[b, pl.ds(base, _PB)], xo_own)
        pltpu.sync_copy(x1_hbm.at[b, pl.ds(base, _PB)], D)
        pltpu.sync_copy(D, X1.at[pl.ds(base, _PB)])
        plsc.subcore_barrier()

        def row1(i, carry):
            for k in range(8):
                off = i * _CH + k * 16
                xh = D[pl.ds(off, 16)]
                m0v = xo_own[pl.ds(off, 16)] > 0
                adjh = jnp.where(xh >= _N, xh - _N, xh)
                adjh = jnp.minimum(adjh, _N - 1)
                owni = _iota16() + off
                ci[i, pl.ds(k * 16, 16)] = jnp.where(m0v, adjh, owni)
            return carry
        lax.fori_loop(0, _NCH, row1, 0, unroll=False)

        _gather_chunks(X1, ci, Bb, sem, True)

        def row2(i, ts):
            for k in range(8):
                off = i * _CH + k * 16
                m0v = xo_own[pl.ds(off, 16)] > 0
                out = jnp.where(m0v, Bb[pl.ds(off, 16)], D[pl.ds(off, 16)])
                D[pl.ds(off, 16)] = out
                diff = jnp.abs(xf_own[pl.ds(off, 16)] - out)
                ts = ts + jnp.where(m0v, diff, 0).astype(jnp.int32)
            return ts
        tsv = lax.fori_loop(0, _NCH, row2, jnp.zeros((16,), jnp.int32),
                            unroll=False)

        pltpu.sync_copy(D, xfo_hbm.at[b, pl.ds(base, _PB)])
        cbuf[...] = tsv
        pltpu.sync_copy(cbuf, tsk_hbm.at[b, pl.ds(s * 16, 16)])
        return carry

    lax.fori_loop(0, _NBPC, batch_body, 0, unroll=False)


def _mesh():
    return plsc.VectorSubcoreMesh(core_axis_name="c", subcore_axis_name="s")


_SC_SCRATCH = [
    pltpu.VMEM_SHARED((_N,), jnp.int32),
    pltpu.VMEM_SHARED((_N + _PB + 16,), jnp.int32),
    pltpu.VMEM_SHARED((_NS, 16), jnp.int32),
    pltpu.VMEM((_PB,), jnp.int32),
    pltpu.VMEM((_PB,), jnp.int32),
    pltpu.VMEM((_PB,), jnp.int32),
    pltpu.VMEM((_PB,), jnp.int32),
    pltpu.VMEM((_NCH, _CH), jnp.int32),
    pltpu.VMEM((16,), jnp.int32),
    pltpu.VMEM((16,), jnp.int32),
    pltpu.VMEM((_NS, 16), jnp.int32),
    pltpu.VMEM((16,), jnp.int32),
    pltpu.SemaphoreType.DMA,
]


def _s1(xf, x2f, xo, vv):
    f = pl.kernel(
        _s1_body,
        out_type=(jax.ShapeDtypeStruct((_B, _N), jnp.int32),
                  jax.ShapeDtypeStruct((_B, _NS * 16), jnp.int32)),
        mesh=_mesh(),
        scratch_types=list(_SC_SCRATCH),
    )
    return f(xf, x2f, xo, vv)


def _sfix(xf, x2f, xo, vv, x1h):
    f = pl.kernel(
        _sfix_body,
        out_type=(jax.ShapeDtypeStruct((_B, _N), jnp.int32),
                  jax.ShapeDtypeStruct((_B, _NS * 16), jnp.int32)),
        mesh=_mesh(),
        scratch_types=list(_SC_SCRATCH),
    )
    return f(xf, x2f, xo, vv, x1h)


def _s2(xf, xo, x1h):
    f = pl.kernel(
        _s2_body,
        out_type=(jax.ShapeDtypeStruct((_B, _N), jnp.int32),
                  jax.ShapeDtypeStruct((_B, _NS * 16), jnp.int32)),
        mesh=_mesh(),
        scratch_types=[
            pltpu.VMEM_SHARED((_N,), jnp.int32),
            pltpu.VMEM((_PB,), jnp.int32),
            pltpu.VMEM((_PB,), jnp.int32),
            pltpu.VMEM((_NCH, _CH), jnp.int32),
            pltpu.VMEM((_PB,), jnp.int32),
            pltpu.VMEM((_PB,), jnp.int32),
            pltpu.VMEM((16,), jnp.int32),
            pltpu.SemaphoreType.DMA,
        ],
    )
    return f(xf, xo, x1h)


def kernel(x):
    x = x.astype(jnp.int32)
    B, C, H, W = x.shape
    xi = x.reshape(B, H, W)
    xf = _tc_prologue(xi).reshape(B, _N)
    xo = xi.reshape(B, _N)
    T_skip = jnp.ones((B,), dtype=jnp.int32)
    cum = jnp.ones((B,), dtype=jnp.int32)
    it = jnp.array(_MAX_ITERATIONS, dtype=jnp.int32)

    def loop_cond(carry):
        xf, cum, T_skip, it = carry
        return (jnp.sum(T_skip) > 0) & (it > 0)

    def loop_body(carry):
        xf, cum, T_skip, it = carry
        it = it - 1
        cum = cum & (T_skip > 0).astype(jnp.int32)
        x2f = _tc_stencil(xf.reshape(B, H, W), xi).reshape(B, _N)
        vv = jnp.broadcast_to(cum[:, None], (B, 16)).astype(jnp.int32)
        x1h, flg = _s1(xf, x2f, xo, vv)

        def fix_cond(c):
            return jnp.sum(jnp.max(c[1], axis=1)) > 0

        def fix_body(c):
            return _sfix(xf, x2f, xo, vv, c[0])

        x1h, flg = lax.while_loop(fix_cond, fix_body, (x1h, flg))
        xf2, tsk = _s2(xf, xo, x1h)
        T_skip = jnp.sum(tsk, axis=1).astype(jnp.int32)
        return xf2, cum, T_skip, it

    xf, cum, T_skip, it = jax.lax.while_loop(
        loop_cond, loop_body, (xf, cum, T_skip, it))
    return xf.reshape(B, 1, H, W)
